# baseline (speedup 1.0000x reference)
import functools

import jax
import jax.numpy as jnp
from jax import lax
from jax.experimental import pallas as pl
from jax.experimental.pallas import tpu as pltpu
from jax.experimental.pallas import tpu_sc as plsc

_NUM_FIELDS = 26
_PER_FIELD = 100000
_EMB_DIM = 32
_ROWS = _NUM_FIELDS * _PER_FIELD
_NW = 32
_ROWS_PER_DMA = 128
_LANES = 16
_NBUF = 8

_FULL_COLS = _ROWS // 128
_REM = _ROWS - _FULL_COLS * 128
_GCOLS = 4
_GW = _GCOLS * 128
_NGROUPS = _FULL_COLS // _GCOLS
_GROUPS_PER_W = -(-_NGROUPS // _NW)
_RDEPTH = 3
_PADW = _EMB_DIM + 8


def _relayout(table_t, rem_flat):
    mesh = plsc.VectorSubcoreMesh(core_axis_name="c", subcore_axis_name="s")

    @functools.partial(
        pl.kernel,
        out_type=jax.ShapeDtypeStruct((_ROWS * _EMB_DIM,), jnp.float32),
        mesh=mesh,
        scratch_types=(
            [pltpu.VMEM((_EMB_DIM, _GW + 8), jnp.float32)] * _RDEPTH
            + [pltpu.VMEM((_GW * _EMB_DIM,), jnp.float32)] * _RDEPTH
            + [pltpu.VMEM((_REM * _EMB_DIM,), jnp.float32)]
            + [pltpu.SemaphoreType.DMA] * (2 * _RDEPTH)
        ),
        compiler_params=pltpu.CompilerParams(needs_layout_passes=False),
    )
    def _k(tt_hbm, rem_hbm, out_hbm, *bufs):
        bins = bufs[:_RDEPTH]
        touts = bufs[_RDEPTH:2 * _RDEPTH]
        rstage = bufs[2 * _RDEPTH]
        sis = bufs[2 * _RDEPTH + 1:3 * _RDEPTH + 1]
        sos = bufs[3 * _RDEPTH + 1:]

        wid = lax.axis_index("s") * 2 + lax.axis_index("c")
        start = wid * _GROUPS_PER_W
        count = jnp.minimum(_GROUPS_PER_W, jnp.maximum(_NGROUPS - start, 0))
        lane = lax.iota(jnp.int32, _LANES)
        kv0 = lane
        kv1 = lane + _LANES

        def in_desc(g, p):
            src = tt_hbm.at[:, pl.ds(pl.multiple_of(g * _GW, 128), _GW)]
            return pltpu.make_async_copy(src, bins[p].at[:, pl.ds(0, _GW)],
                                         sis[p])

        def out_desc(g, p):
            dst = out_hbm.at[pl.ds(pl.multiple_of(g * _GW * _EMB_DIM, 8),
                                   _GW * _EMB_DIM)]
            return pltpu.make_async_copy(touts[p], dst, sos[p])

        def transpose_block(p):
            @plsc.parallel_loop(0, _GW, unroll=4)
            def tr_body(l):
                lv = jnp.broadcast_to(l, (_LANES,))
                base = jnp.broadcast_to(l * _EMB_DIM, (_LANES,)) + lane
                v0 = plsc.load_gather(bins[p], [kv0, lv])
                plsc.store_scatter(touts[p], [base], v0)
                v1 = plsc.load_gather(bins[p], [kv1, lv])
                plsc.store_scatter(touts[p], [base + _LANES], v1)

        for p in range(_RDEPTH):
            @pl.when(p < count)
            def _():
                in_desc(start + p, p).start()

        def ring_body(t, carry):
            for p in range(_RDEPTH):
                i = t * _RDEPTH + p

                @pl.when(i < count)
                def _():
                    g = start + i
                    in_desc(g, p).wait()

                    @pl.when(i >= _RDEPTH)
                    def _():
                        out_desc(g - _RDEPTH, p).wait()

                    transpose_block(p)
                    out_desc(g, p).start()

                    @pl.when(i + _RDEPTH < count)
                    def _():
                        in_desc(g + _RDEPTH, p).start()

            return carry

        lax.fori_loop(0, (count + _RDEPTH - 1) // _RDEPTH, ring_body, 0)

        for p in range(_RDEPTH):
            last_p = count - 1 - jnp.mod(count - 1 - p, _RDEPTH)

            @pl.when(last_p >= 0)
            def _():
                out_desc(start + last_p, p).wait()

        @pl.when(wid == _NW - 1)
        def _():
            pltpu.sync_copy(rem_hbm, rstage)
            dst = out_hbm.at[pl.ds(_FULL_COLS * 128 * _EMB_DIM,
                                   _REM * _EMB_DIM)]
            pltpu.sync_copy(rstage, dst)

    return _k(table_t, rem_flat)


def _gather(x_flat, table_rows):
    N = x_flat.shape[0]
    per_w = N // _NW
    n_dma = per_w // _ROWS_PER_DMA
    ngroups = n_dma // _NBUF
    mesh = plsc.VectorSubcoreMesh(core_axis_name="c", subcore_axis_name="s")

    @functools.partial(
        pl.kernel,
        out_type=jax.ShapeDtypeStruct((N, _EMB_DIM), jnp.float32),
        mesh=mesh,
        scratch_types=(
            [pltpu.VMEM((per_w,), jnp.int32)]
            + [pltpu.VMEM((_ROWS_PER_DMA, _EMB_DIM), jnp.float32)] * _NBUF
            + [pltpu.SemaphoreType.DMA] * (2 * _NBUF)
        ),
        compiler_params=pltpu.CompilerParams(use_tc_tiling_on_sc=False),
    )
    def _k(x_hbm, table_hbm, out_hbm, idx_v, *bufs):
        rows = bufs[:_NBUF]
        gsem = bufs[_NBUF:2 * _NBUF]
        wsem = bufs[2 * _NBUF:]

        wid = lax.axis_index("s") * 2 + lax.axis_index("c")
        base = pl.multiple_of(wid * per_w, 8)
        pltpu.sync_copy(x_hbm.at[pl.ds(base, per_w)], idx_v)

        lane = lax.iota(jnp.int32, _LANES)

        @plsc.parallel_loop(0, per_w // _LANES, unroll=4)
        def add_body(i):
            col = pl.multiple_of(i * _LANES, _LANES)
            f = lax.rem(col + lane, _NUM_FIELDS)
            idx_v[pl.ds(col, _LANES)] = idx_v[pl.ds(col, _LANES)] + f * _PER_FIELD

        def gather_desc(r, b):
            off = pl.multiple_of(r * _ROWS_PER_DMA, 8)
            idx_slice = idx_v.at[pl.ds(off, _ROWS_PER_DMA)]
            return pltpu.make_async_copy(table_hbm.at[idx_slice], rows[b], gsem[b])

        def write_desc(r, b):
            out_off = pl.multiple_of(base + r * _ROWS_PER_DMA, 8)
            return pltpu.make_async_copy(
                rows[b], out_hbm.at[pl.ds(out_off, _ROWS_PER_DMA)], wsem[b])

        for b in range(_NBUF):
            gather_desc(b, b).start()

        def group_body(g, carry):
            rbase = g * _NBUF
            for b in range(_NBUF):
                gather_desc(rbase + b, b).wait()
                write_desc(rbase + b, b).start()
            for b in range(_NBUF):
                write_desc(rbase + b, b).wait()

                @pl.when(g < ngroups - 1)
                def _():
                    gather_desc(rbase + _NBUF + b, b).start()

            return carry

        lax.fori_loop(0, ngroups, group_body, 0)

    return _k(x_flat, table_rows)


def kernel(x, table):
    B, F = x.shape
    N = B * F
    rem = lax.slice(table, (_FULL_COLS * 128, 0), (_ROWS, _EMB_DIM)).reshape(-1)
    rows1d = _relayout(table.T, rem)
    out = _gather(x.reshape(N), rows1d.reshape(_ROWS, _EMB_DIM))
    return out.reshape(B, F, _EMB_DIM)


# speedup vs baseline: 2.1629x; 2.1629x over previous
import functools

import jax
import jax.numpy as jnp
from jax import lax
from jax.experimental import pallas as pl
from jax.experimental.pallas import tpu as pltpu
from jax.experimental.pallas import tpu_sc as plsc

_NUM_FIELDS = 26
_PER_FIELD = 100000
_EMB_DIM = 32
_ROWS = _NUM_FIELDS * _PER_FIELD
_NW = 32
_ROWS_PER_DMA = 128
_LANES = 16
_NBUF = 8

_FULL_COLS = _ROWS // 128
_REM = _ROWS - _FULL_COLS * 128
_GCOLS = 4
_GW = _GCOLS * 128
_NGROUPS = _FULL_COLS // _GCOLS
_GROUPS_PER_W = -(-_NGROUPS // _NW)
_RDEPTH = 3
_PADW = _EMB_DIM + 8


def _relayout(table_t, rem_flat):
    mesh = plsc.VectorSubcoreMesh(core_axis_name="c", subcore_axis_name="s")

    @functools.partial(
        pl.kernel,
        out_type=jax.ShapeDtypeStruct((_ROWS * _PADW,), jnp.float32),
        mesh=mesh,
        scratch_types=(
            [pltpu.VMEM((_EMB_DIM, _GW + 8), jnp.float32)] * _RDEPTH
            + [pltpu.VMEM((_GW * _PADW,), jnp.float32)] * _RDEPTH
            + [pltpu.VMEM((_REM * _EMB_DIM,), jnp.float32)]
            + [pltpu.SemaphoreType.DMA] * (2 * _RDEPTH)
        ),
        compiler_params=pltpu.CompilerParams(needs_layout_passes=False),
    )
    def _k(tt_hbm, rem_hbm, out_hbm, *bufs):
        bins = bufs[:_RDEPTH]
        touts = bufs[_RDEPTH:2 * _RDEPTH]
        rstage = bufs[2 * _RDEPTH]
        sis = bufs[2 * _RDEPTH + 1:3 * _RDEPTH + 1]
        sos = bufs[3 * _RDEPTH + 1:]

        wid = lax.axis_index("s") * 2 + lax.axis_index("c")
        start = wid * _GROUPS_PER_W
        count = jnp.minimum(_GROUPS_PER_W, jnp.maximum(_NGROUPS - start, 0))
        lane = lax.iota(jnp.int32, _LANES)
        njv = _GW // _LANES
        colvs = [lane + j * _LANES for j in range(njv)]
        colvs40 = [(lane + j * _LANES) * _PADW for j in range(njv)]

        def in_desc(g, p):
            src = tt_hbm.at[:, pl.ds(pl.multiple_of(g * _GW, 128), _GW)]
            return pltpu.make_async_copy(src, bins[p].at[:, pl.ds(0, _GW)],
                                         sis[p])

        def out_desc(g, p):
            dst = out_hbm.at[pl.ds(pl.multiple_of(g * _GW * _PADW, 8),
                                   _GW * _PADW)]
            return pltpu.make_async_copy(touts[p], dst, sos[p])

        def transpose_block(p):
            @plsc.parallel_loop(0, _EMB_DIM, unroll=4)
            def tr_body(k):
                kv = jnp.broadcast_to(k, (_LANES,))
                for j in range(njv):
                    v = plsc.load_gather(bins[p], [kv, colvs[j]])
                    plsc.store_scatter(touts[p], [colvs40[j] + k], v)

        for p in range(_RDEPTH):
            @pl.when(p < count)
            def _():
                in_desc(start + p, p).start()

        def ring_body(t, carry):
            for p in range(_RDEPTH):
                i = t * _RDEPTH + p

                @pl.when(i < count)
                def _():
                    g = start + i
                    in_desc(g, p).wait()

                    @pl.when(i >= _RDEPTH)
                    def _():
                        out_desc(g - _RDEPTH, p).wait()

                    transpose_block(p)
                    out_desc(g, p).start()

                    @pl.when(i + _RDEPTH < count)
                    def _():
                        in_desc(g + _RDEPTH, p).start()

            return carry

        lax.fori_loop(0, (count + _RDEPTH - 1) // _RDEPTH, ring_body, 0)

        for p in range(_RDEPTH):
            last_p = count - 1 - jnp.mod(count - 1 - p, _RDEPTH)

            @pl.when(last_p >= 0)
            def _():
                out_desc(start + last_p, p).wait()

        @pl.when(wid == _NW - 1)
        def _():
            pltpu.sync_copy(rem_hbm, rstage)

            def rem_body(l, carry):
                touts[0][pl.ds(l * _PADW, _LANES)] = rstage[pl.ds(l * _EMB_DIM, _LANES)]
                touts[0][pl.ds(l * _PADW + _LANES, _LANES)] = (
                    rstage[pl.ds(l * _EMB_DIM + _LANES, _LANES)])
                return carry

            lax.fori_loop(0, _REM, rem_body, 0)
            dst = out_hbm.at[pl.ds(_FULL_COLS * 128 * _PADW, _REM * _PADW)]
            pltpu.sync_copy(touts[0].at[pl.ds(0, _REM * _PADW)], dst)

    return _k(table_t, rem_flat)


def _gather(x_flat, table_rows):
    N = x_flat.shape[0]
    per_w = N // _NW
    n_dma = per_w // _ROWS_PER_DMA
    ngroups = n_dma // _NBUF
    mesh = plsc.VectorSubcoreMesh(core_axis_name="c", subcore_axis_name="s")

    @functools.partial(
        pl.kernel,
        out_type=jax.ShapeDtypeStruct((N, _EMB_DIM), jnp.float32),
        mesh=mesh,
        scratch_types=(
            [pltpu.VMEM((per_w,), jnp.int32)]
            + [pltpu.VMEM((_ROWS_PER_DMA, _PADW), jnp.float32)] * _NBUF
            + [pltpu.SemaphoreType.DMA] * (2 * _NBUF)
        ),
        compiler_params=pltpu.CompilerParams(use_tc_tiling_on_sc=False),
    )
    def _k(x_hbm, table_hbm, out_hbm, idx_v, *bufs):
        rows = bufs[:_NBUF]
        gsem = bufs[_NBUF:2 * _NBUF]
        wsem = bufs[2 * _NBUF:]

        wid = lax.axis_index("s") * 2 + lax.axis_index("c")
        base = pl.multiple_of(wid * per_w, 8)
        pltpu.sync_copy(x_hbm.at[pl.ds(base, per_w)], idx_v)

        lane = lax.iota(jnp.int32, _LANES)

        @plsc.parallel_loop(0, per_w // _LANES, unroll=4)
        def add_body(i):
            col = pl.multiple_of(i * _LANES, _LANES)
            f = lax.rem(col + lane, _NUM_FIELDS)
            idx_v[pl.ds(col, _LANES)] = idx_v[pl.ds(col, _LANES)] + f * _PER_FIELD

        def gather_desc(r, b):
            off = pl.multiple_of(r * _ROWS_PER_DMA, 8)
            idx_slice = idx_v.at[pl.ds(off, _ROWS_PER_DMA)]
            return pltpu.make_async_copy(table_hbm.at[idx_slice], rows[b], gsem[b])

        def write_desc(r, b):
            out_off = pl.multiple_of(base + r * _ROWS_PER_DMA, 8)
            return pltpu.make_async_copy(
                rows[b].at[:, pl.ds(0, _EMB_DIM)],
                out_hbm.at[pl.ds(out_off, _ROWS_PER_DMA)], wsem[b])

        for b in range(_NBUF):
            gather_desc(b, b).start()

        def group_body(g, carry):
            rbase = g * _NBUF
            for b in range(_NBUF):
                gather_desc(rbase + b, b).wait()
                write_desc(rbase + b, b).start()
            for b in range(_NBUF):
                write_desc(rbase + b, b).wait()

                @pl.when(g < ngroups - 1)
                def _():
                    gather_desc(rbase + _NBUF + b, b).start()

            return carry

        lax.fori_loop(0, ngroups, group_body, 0)

    return _k(x_flat, table_rows)


def kernel(x, table):
    B, F = x.shape
    N = B * F
    rem = lax.slice(table, (_FULL_COLS * 128, 0), (_ROWS, _EMB_DIM)).reshape(-1)
    rows_pad = _relayout(table.T, rem)
    out = _gather(x.reshape(N), rows_pad.reshape(_ROWS, _PADW))
    return out.reshape(B, F, _EMB_DIM)
